# baseline (speedup 1.0000x reference)
import jax
import jax.numpy as jnp
from jax import lax
from jax.experimental import pallas as pl
from jax.experimental.pallas import tpu as pltpu
from jax.experimental.pallas import tpu_sc as plsc

E = 320000
D_FEAT = 128
D_EDGE = 16
D_OUT = 17

NC = 2
NS = 16
NW = NC * NS
ROWS_PER_W = E // NW
B = 2000
NCHUNK = ROWS_PER_W // B
L = 16
NGRP = B // L


def _edge_update(vattr_flat_hbm, edge_attr_flat_hbm, out_flat_hbm,
                 ea_v, col_v, idx_v, out_v, sem):
    wid = lax.axis_index("s") * NC + lax.axis_index("c")
    base0 = wid * ROWS_PER_W
    lanes = lax.iota(jnp.int32, L)

    def chunk_body(i, carry):
        base = base0 + i * B

        pltpu.sync_copy(edge_attr_flat_hbm.at[pl.ds(base * D_EDGE, B * D_EDGE)],
                        ea_v)

        def idx_body(g, c):
            idx_v[pl.ds(g * L, L)] = (base + g * L + lanes) * D_FEAT + 1
            return c
        lax.fori_loop(0, NGRP, idx_body, 0)
        gather = pltpu.async_copy(vattr_flat_hbm.at[idx_v], col_v, sem)

        def row_body(r, c):
            out_v[pl.ds(r * D_OUT, D_EDGE)] = ea_v[pl.ds(r * D_EDGE, D_EDGE)]
            return c
        lax.fori_loop(0, B, row_body, 0)
        gather.wait()

        def scat_body(g, c):
            vals = col_v[pl.ds(g * L, L)]
            dst = (g * L + lanes) * D_OUT + D_EDGE
            plsc.store_scatter(out_v, [dst], vals)
            return c
        lax.fori_loop(0, NGRP, scat_body, 0)

        pltpu.sync_copy(out_v, out_flat_hbm.at[pl.ds(base * D_OUT, B * D_OUT)])
        return carry

    lax.fori_loop(0, NCHUNK, chunk_body, 0)


def kernel(vattr_i, vattr_j, edge_attr, g, batch):
    k = pl.kernel(
        _edge_update,
        out_type=jax.ShapeDtypeStruct((E * D_OUT,), jnp.float32),
        mesh=plsc.VectorSubcoreMesh(core_axis_name="c", subcore_axis_name="s"),
        scratch_types=[
            pltpu.VMEM((B * D_EDGE,), jnp.float32),
            pltpu.VMEM((B,), jnp.float32),
            pltpu.VMEM((B,), jnp.int32),
            pltpu.VMEM((B * D_OUT,), jnp.float32),
            pltpu.SemaphoreType.DMA,
        ],
        compiler_params=pltpu.CompilerParams(needs_layout_passes=False),
    )
    out_flat = k(vattr_j.reshape(E * D_FEAT), edge_attr.reshape(E * D_EDGE))
    return out_flat.reshape(E, D_OUT)


# speedup vs baseline: 1.0689x; 1.0689x over previous
import jax
import jax.numpy as jnp
from jax import lax
from jax.experimental import pallas as pl
from jax.experimental.pallas import tpu as pltpu
from jax.experimental.pallas import tpu_sc as plsc

E = 320000
D_FEAT = 128
D_EDGE = 16
D_OUT = 17

NC = 2
NS = 16
NW = NC * NS
ROWS_PER_W = E // NW
B = 2000
NCHUNK = ROWS_PER_W // B
L = 16
NGRP = B // L


def _edge_update(vattr_flat_hbm, edge_attr_flat_hbm, out_flat_hbm,
                 ea_v, col_v, idx_v, out_v, sem):
    wid = lax.axis_index("s") * NC + lax.axis_index("c")
    base0 = wid * ROWS_PER_W
    lanes = lax.iota(jnp.int32, L)

    @plsc.parallel_loop(0, NGRP, unroll=8)
    def _idx(g):
        idx_v[pl.ds(g * L, L)] = (g * L + lanes) * D_FEAT + 1

    def chunk_body(i, carry):
        base = base0 + i * B

        pltpu.sync_copy(edge_attr_flat_hbm.at[pl.ds(base * D_EDGE, B * D_EDGE)],
                        ea_v)

        gather = pltpu.async_copy(
            vattr_flat_hbm.at[pl.ds(base * D_FEAT, B * D_FEAT)].at[idx_v],
            col_v, sem)

        @plsc.parallel_loop(0, B, unroll=8)
        def _rows(r):
            out_v[pl.ds(r * D_OUT, D_EDGE)] = ea_v[pl.ds(r * D_EDGE, D_EDGE)]

        gather.wait()

        @plsc.parallel_loop(0, NGRP, unroll=8)
        def _scat(g):
            vals = col_v[pl.ds(g * L, L)]
            dst = (g * L + lanes) * D_OUT + D_EDGE
            plsc.store_scatter(out_v, [dst], vals)

        pltpu.sync_copy(out_v, out_flat_hbm.at[pl.ds(base * D_OUT, B * D_OUT)])
        return carry

    lax.fori_loop(0, NCHUNK, chunk_body, 0)


def kernel(vattr_i, vattr_j, edge_attr, g, batch):
    k = pl.kernel(
        _edge_update,
        out_type=jax.ShapeDtypeStruct((E * D_OUT,), jnp.float32),
        mesh=plsc.VectorSubcoreMesh(core_axis_name="c", subcore_axis_name="s"),
        scratch_types=[
            pltpu.VMEM((B * D_EDGE,), jnp.float32),
            pltpu.VMEM((B,), jnp.float32),
            pltpu.VMEM((B,), jnp.int32),
            pltpu.VMEM((B * D_OUT,), jnp.float32),
            pltpu.SemaphoreType.DMA,
        ],
        compiler_params=pltpu.CompilerParams(needs_layout_passes=False),
    )
    out_flat = k(vattr_j.reshape(E * D_FEAT), edge_attr.reshape(E * D_EDGE))
    return out_flat.reshape(E, D_OUT)
